# baseline (speedup 1.0000x reference)
import functools

import jax
import jax.numpy as jnp
from jax import lax
from jax.experimental import pallas as pl
from jax.experimental.pallas import tpu as pltpu
from jax.experimental.pallas import tpu_sc as plsc

_LANES = 128


def _make_gather(B, V, D, NC, NS):
    NW = NC * NS
    b_per_w = B // NW
    CHUNK = 128
    n_steps = b_per_w // CHUNK
    G = 128
    n_g = CHUNK // G
    NBUF = 4
    assert b_per_w % CHUNK == 0 and CHUNK % G == 0 and n_steps > NBUF

    mesh = plsc.VectorSubcoreMesh(core_axis_name="c", subcore_axis_name="s")

    @functools.partial(
        pl.kernel,
        mesh=mesh,
        out_type=jax.ShapeDtypeStruct((B, _LANES), jnp.float32),
        scratch_types=[
            pltpu.VMEM((b_per_w,), jnp.int32),
            pltpu.VMEM((NBUF, CHUNK, _LANES), jnp.float32),
            [pltpu.SemaphoreType.DMA] * NBUF,
            [pltpu.SemaphoreType.DMA] * NBUF,
        ],
    )
    def gather_kernel(idx_hbm, tab_hbm, out_hbm, idx_v, rows_v, gsems, osems):
        wid = lax.axis_index("s") * NC + lax.axis_index("c")
        base = wid * b_per_w

        pltpu.sync_copy(idx_hbm.at[pl.ds(base, b_per_w)], idx_v)

        def fire(step, slot):
            for j in range(n_g):
                pltpu.async_copy(
                    tab_hbm.at[idx_v.at[pl.ds(step * CHUNK + j * G, G)]],
                    rows_v.at[slot, pl.ds(j * G, G)],
                    gsems[slot],
                )

        def drain_gathers(step, slot):
            for j in range(n_g):
                pltpu.make_async_copy(
                    tab_hbm.at[idx_v.at[pl.ds(step * CHUNK + j * G, G)]],
                    rows_v.at[slot, pl.ds(j * G, G)],
                    gsems[slot],
                ).wait()

        def fire_write(step, slot):
            off = base + step * CHUNK
            pltpu.async_copy(rows_v.at[slot], out_hbm.at[pl.ds(off, CHUNK)],
                             osems[slot])

        def wait_write(step, slot):
            off = base + step * CHUNK
            pltpu.make_async_copy(rows_v.at[slot],
                                  out_hbm.at[pl.ds(off, CHUNK)],
                                  osems[slot]).wait()

        for s in range(NBUF - 1):
            fire(s, s)

        def body(i, carry):
            s_cur = lax.rem(i, NBUF)
            s_pre = lax.rem(i + NBUF - 1, NBUF)
            for s in range(NBUF):
                @pl.when(s_cur == s)
                def _():
                    drain_gathers(i, s)
                    fire_write(i, s)

                @pl.when(jnp.logical_and(s_pre == s, i + NBUF - 1 < n_steps))
                def _():
                    @pl.when(i >= 1)
                    def _():
                        wait_write(i - 1, s)
                    fire(i + NBUF - 1, s)
            return carry

        lax.fori_loop(0, n_steps, body, 0)

        for k in range(NBUF):
            step = n_steps - NBUF + k
            wait_write(step, step % NBUF)

    return gather_kernel


def kernel(x, table):
    Bb, S = x.shape
    V, D = table.shape
    B = Bb * S
    info = plsc.get_sparse_core_info()
    gather = _make_gather(B, V, D, info.num_cores, info.num_subcores)
    tab_pad = jnp.pad(table, ((0, 0), (0, _LANES - D)))
    out = gather(x.reshape(B), tab_pad)
    return out[:, :D].reshape(Bb, S, D)


# speedup vs baseline: 1.0015x; 1.0015x over previous
import functools

import jax
import jax.numpy as jnp
from jax import lax
from jax.experimental import pallas as pl
from jax.experimental.pallas import tpu as pltpu
from jax.experimental.pallas import tpu_sc as plsc

_LANES = 128


def _make_gather(B, V, D, NC, NS):
    NW = NC * NS
    b_per_w = B // NW
    CHUNK = 64
    n_steps = b_per_w // CHUNK
    G = 64
    n_g = CHUNK // G
    NBUF = 8
    assert b_per_w % CHUNK == 0 and CHUNK % G == 0 and n_steps > NBUF

    mesh = plsc.VectorSubcoreMesh(core_axis_name="c", subcore_axis_name="s")

    @functools.partial(
        pl.kernel,
        mesh=mesh,
        out_type=jax.ShapeDtypeStruct((B, _LANES), jnp.float32),
        scratch_types=[
            pltpu.VMEM((b_per_w,), jnp.int32),
            pltpu.VMEM((NBUF, CHUNK, _LANES), jnp.float32),
            [pltpu.SemaphoreType.DMA] * NBUF,
            [pltpu.SemaphoreType.DMA] * NBUF,
        ],
    )
    def gather_kernel(idx_hbm, tab_hbm, out_hbm, idx_v, rows_v, gsems, osems):
        wid = lax.axis_index("s") * NC + lax.axis_index("c")
        base = wid * b_per_w

        pltpu.sync_copy(idx_hbm.at[pl.ds(base, b_per_w)], idx_v)

        def fire(step, slot):
            for j in range(n_g):
                pltpu.async_copy(
                    tab_hbm.at[idx_v.at[pl.ds(step * CHUNK + j * G, G)]],
                    rows_v.at[slot, pl.ds(j * G, G)],
                    gsems[slot],
                )

        def drain_gathers(step, slot):
            for j in range(n_g):
                pltpu.make_async_copy(
                    tab_hbm.at[idx_v.at[pl.ds(step * CHUNK + j * G, G)]],
                    rows_v.at[slot, pl.ds(j * G, G)],
                    gsems[slot],
                ).wait()

        def fire_write(step, slot):
            off = base + step * CHUNK
            pltpu.async_copy(rows_v.at[slot], out_hbm.at[pl.ds(off, CHUNK)],
                             osems[slot])

        def wait_write(step, slot):
            off = base + step * CHUNK
            pltpu.make_async_copy(rows_v.at[slot],
                                  out_hbm.at[pl.ds(off, CHUNK)],
                                  osems[slot]).wait()

        for s in range(NBUF - 1):
            fire(s, s)

        def body(i, carry):
            s_cur = lax.rem(i, NBUF)
            s_pre = lax.rem(i + NBUF - 1, NBUF)
            for s in range(NBUF):
                @pl.when(s_cur == s)
                def _():
                    drain_gathers(i, s)
                    fire_write(i, s)

                @pl.when(jnp.logical_and(s_pre == s, i + NBUF - 1 < n_steps))
                def _():
                    @pl.when(i >= 1)
                    def _():
                        wait_write(i - 1, s)
                    fire(i + NBUF - 1, s)
            return carry

        lax.fori_loop(0, n_steps, body, 0)

        for k in range(NBUF):
            step = n_steps - NBUF + k
            wait_write(step, step % NBUF)

    return gather_kernel


def kernel(x, table):
    Bb, S = x.shape
    V, D = table.shape
    B = Bb * S
    info = plsc.get_sparse_core_info()
    gather = _make_gather(B, V, D, info.num_cores, info.num_subcores)
    tab_pad = jnp.pad(table, ((0, 0), (0, _LANES - D)))
    out = gather(x.reshape(B), tab_pad)
    return out[:, :D].reshape(Bb, S, D)
